# Initial kernel scaffold; baseline (speedup 1.0000x reference)
import functools
import math

import jax
import jax.numpy as jnp
from jax import lax
from jax.experimental import pallas as pl
from jax.experimental.pallas import tpu as pltpu
from jax.experimental.pallas import tpu_sc as plsc

_LAMDA = 0.5
_ALPHA = 0.1
_L = 4

_D = 128
_NW = 32
_NSUB = 16
_CH = 128
_BM = 1000


def _spmm_sc(h, src3, dst3, w3, n_nodes):
    nch = src3.shape[1]
    rpt = n_nodes // _NSUB
    assert rpt * _NSUB == n_nodes, "n_nodes must divide evenly over subcores"
    mesh = plsc.VectorSubcoreMesh(core_axis_name="c", subcore_axis_name="s")

    @functools.partial(
        pl.kernel,
        mesh=mesh,
        out_type=jax.ShapeDtypeStruct((2, n_nodes, _D), jnp.float32),
        scratch_types=[
            pltpu.VMEM((nch, _CH), jnp.int32),
            pltpu.VMEM((nch, _CH), jnp.int32),
            pltpu.VMEM((nch, _CH), jnp.float32),
            pltpu.VMEM((_CH, _D), jnp.float32),
            pltpu.VMEM_SHARED((n_nodes, _D), jnp.float32),
            pltpu.SemaphoreType.DMA,
        ],
    )
    def k(h_hbm, src_hbm, dst_hbm, w_hbm, out_hbm,
          src_v, dst_v, w_v, rows_v, acc_sh, sem):
        c = lax.axis_index("c")
        s = lax.axis_index("s")
        wid = c * _NSUB + s

        def zrow(r, carry):
            for t in range(_D // 16):
                rows_v[r, pl.ds(t * 16, 16)] = jnp.zeros((16,), jnp.float32)
            return carry
        lax.fori_loop(0, _CH, zrow, 0)
        full, tail = divmod(rpt, _CH)
        for t in range(full):
            pltpu.sync_copy(rows_v, acc_sh.at[pl.ds(s * rpt + t * _CH, _CH)])
        if tail:
            pltpu.sync_copy(rows_v.at[pl.ds(0, tail)],
                            acc_sh.at[pl.ds(s * rpt + full * _CH, tail)])
        plsc.subcore_barrier()

        pltpu.sync_copy(src_hbm.at[wid], src_v)
        pltpu.sync_copy(dst_hbm.at[wid], dst_v)
        pltpu.sync_copy(w_hbm.at[wid], w_v)

        def chunk(j, carry):
            pltpu.async_copy(h_hbm.at[src_v.at[j]], rows_v, sem).wait()

            def edge(e, carry2):
                w = w_v[j, e]
                for t in range(_D // 16):
                    idx = (e, pl.ds(t * 16, 16))
                    rows_v[idx] = rows_v[idx] * w
                return carry2
            lax.fori_loop(0, _CH, edge, 0)

            pltpu.sync_copy(rows_v, acc_sh.at[dst_v.at[j]], add=True)
            return carry
        lax.fori_loop(0, nch, chunk, 0)

        plsc.subcore_barrier()
        pltpu.sync_copy(acc_sh.at[pl.ds(s * rpt, rpt)],
                        out_hbm.at[c, pl.ds(s * rpt, rpt)])

    return k(h, src3, dst3, w3)


def _mlp0_tc(x, w, b):
    m = x.shape[0]

    def body(x_ref, w_ref, b_ref, o_ref):
        o_ref[...] = jnp.maximum(
            jnp.dot(x_ref[...], w_ref[...],
                    preferred_element_type=jnp.float32) + b_ref[...], 0.0)

    return pl.pallas_call(
        body,
        grid=(m // _BM,),
        in_specs=[
            pl.BlockSpec((_BM, _D), lambda i: (i, 0)),
            pl.BlockSpec((_D, _D), lambda i: (0, 0)),
            pl.BlockSpec((1, _D), lambda i: (0, 0)),
        ],
        out_specs=pl.BlockSpec((_BM, _D), lambda i: (i, 0)),
        out_shape=jax.ShapeDtypeStruct((m, _D), jnp.float32),
    )(x, w, b)


def _layer_tc(parts, h0, w, beta):
    m = h0.shape[0]

    def body(p_ref, h0_ref, w_ref, o_ref):
        sup = ((1.0 - _ALPHA) * (p_ref[0] + p_ref[1])
               + _ALPHA * h0_ref[...])
        t = jnp.dot(sup, w_ref[...], preferred_element_type=jnp.float32)
        o_ref[...] = jnp.maximum((1.0 - beta) * sup + beta * t, 0.0)

    return pl.pallas_call(
        body,
        grid=(m // _BM,),
        in_specs=[
            pl.BlockSpec((2, _BM, _D), lambda i: (0, i, 0)),
            pl.BlockSpec((_BM, _D), lambda i: (i, 0)),
            pl.BlockSpec((_D, _D), lambda i: (0, 0)),
        ],
        out_specs=pl.BlockSpec((_BM, _D), lambda i: (i, 0)),
        out_shape=jax.ShapeDtypeStruct((m, _D), jnp.float32),
    )(parts, h0, w)


def _final_tc(parts, h0, wc, beta, w1, b1):
    m = h0.shape[0]
    d_out = w1.shape[1]

    def body(p_ref, h0_ref, wc_ref, w1_ref, b1_ref, o_ref):
        sup = ((1.0 - _ALPHA) * (p_ref[0] + p_ref[1])
               + _ALPHA * h0_ref[...])
        t = jnp.dot(sup, wc_ref[...], preferred_element_type=jnp.float32)
        h = jnp.maximum((1.0 - beta) * sup + beta * t, 0.0)
        logits = jnp.dot(h, w1_ref[...],
                         preferred_element_type=jnp.float32) + b1_ref[...]
        mx = jnp.max(logits, axis=1, keepdims=True)
        lse = jnp.log(jnp.sum(jnp.exp(logits - mx), axis=1, keepdims=True))
        o_ref[...] = logits - mx - lse

    return pl.pallas_call(
        body,
        grid=(m // _BM,),
        in_specs=[
            pl.BlockSpec((2, _BM, _D), lambda i: (0, i, 0)),
            pl.BlockSpec((_BM, _D), lambda i: (i, 0)),
            pl.BlockSpec((_D, _D), lambda i: (0, 0)),
            pl.BlockSpec((_D, d_out), lambda i: (0, 0)),
            pl.BlockSpec((1, d_out), lambda i: (0, 0)),
        ],
        out_specs=pl.BlockSpec((_BM, d_out), lambda i: (i, 0)),
        out_shape=jax.ShapeDtypeStruct((m, d_out), jnp.float32),
    )(parts, h0, wc, w1, b1)


def kernel(feature, edge_weight, W_fc0, b_fc0, W_conv, W_fc1, b_fc1,
           edge_index):
    n = feature.shape[0]
    e = edge_index.shape[1]

    per_chunk = _NW * _CH
    nch = -(-e // per_chunk)
    e_pad = nch * per_chunk
    ei = edge_index.astype(jnp.int32)
    dst3 = jnp.pad(ei[0], (0, e_pad - e)).reshape(_NW, nch, _CH)
    src3 = jnp.pad(ei[1], (0, e_pad - e)).reshape(_NW, nch, _CH)
    w3 = jnp.pad(edge_weight, (0, e_pad - e)).reshape(_NW, nch, _CH)

    h0 = _mlp0_tc(feature, W_fc0, b_fc0.reshape(1, _D))
    h = h0
    out = None
    for l in range(1, _L + 1):
        beta = math.log(_LAMDA / l + 1.0)
        parts = _spmm_sc(h, src3, dst3, w3, n)
        if l < _L:
            h = _layer_tc(parts, h0, W_conv[l - 1], beta)
        else:
            out = _final_tc(parts, h0, W_conv[l - 1], beta, W_fc1,
                            b_fc1.reshape(1, -1))
    return out


# speedup vs baseline: 4.7383x; 4.7383x over previous
import functools
import math

import jax
import jax.numpy as jnp
from jax import lax
from jax.experimental import pallas as pl
from jax.experimental.pallas import tpu as pltpu
from jax.experimental.pallas import tpu_sc as plsc

_LAMDA = 0.5
_ALPHA = 0.1
_L = 4

_D = 128
_NW = 32
_NSUB = 16
_CH = 128
_BM = 1000


def _spmm_sc(h, src3, dst3, w3, n_nodes):
    nch = src3.shape[1]
    rpt = -(-n_nodes // (_NSUB * _CH)) * _CH
    n_acc = rpt * _NSUB
    mesh = plsc.VectorSubcoreMesh(core_axis_name="c", subcore_axis_name="s")

    @functools.partial(
        pl.kernel,
        mesh=mesh,
        out_type=jax.ShapeDtypeStruct((2, n_acc, _D), jnp.float32),
        scratch_types=[
            pltpu.VMEM((nch, _CH), jnp.int32),
            pltpu.VMEM((nch, _CH), jnp.int32),
            pltpu.VMEM((nch, _CH), jnp.float32),
            pltpu.VMEM((_CH, _D), jnp.float32),
            pltpu.VMEM_SHARED((n_acc, _D), jnp.float32),
            pltpu.SemaphoreType.DMA,
        ],
    )
    def k(h_hbm, src_hbm, dst_hbm, w_hbm, out_hbm,
          src_v, dst_v, w_v, rows_v, acc_sh, sem):
        c = lax.axis_index("c")
        s = lax.axis_index("s")
        wid = c * _NSUB + s

        def zrow(r, carry):
            for t in range(_D // 16):
                rows_v[r, pl.ds(t * 16, 16)] = jnp.zeros((16,), jnp.float32)
            return carry
        lax.fori_loop(0, _CH, zrow, 0)
        for t in range(rpt // _CH):
            pltpu.sync_copy(rows_v, acc_sh.at[pl.ds(s * rpt + t * _CH, _CH)])
        plsc.subcore_barrier()

        pltpu.sync_copy(src_hbm.at[wid], src_v)
        pltpu.sync_copy(dst_hbm.at[wid], dst_v)
        pltpu.sync_copy(w_hbm.at[wid], w_v)

        def chunk(j, carry):
            pltpu.async_copy(h_hbm.at[src_v.at[j]], rows_v, sem).wait()

            def egroup(g, carry2):
                wv = w_v[j, pl.ds(g * 16, 16)]
                for el in range(16):
                    e = g * 16 + el
                    w = wv[el]
                    for t in range(_D // 16):
                        idx = (e, pl.ds(t * 16, 16))
                        rows_v[idx] = rows_v[idx] * w
                return carry2
            lax.fori_loop(0, _CH // 16, egroup, 0)

            pltpu.sync_copy(rows_v, acc_sh.at[dst_v.at[j]], add=True)
            return carry
        lax.fori_loop(0, nch, chunk, 0)

        plsc.subcore_barrier()
        pltpu.sync_copy(acc_sh.at[pl.ds(s * rpt, rpt)],
                        out_hbm.at[c, pl.ds(s * rpt, rpt)])

    return k(h, src3, dst3, w3)


def _mlp0_tc(x, w, b):
    m = x.shape[0]

    def body(x_ref, w_ref, b_ref, o_ref):
        o_ref[...] = jnp.maximum(
            jnp.dot(x_ref[...], w_ref[...],
                    preferred_element_type=jnp.float32) + b_ref[...], 0.0)

    return pl.pallas_call(
        body,
        grid=(m // _BM,),
        in_specs=[
            pl.BlockSpec((_BM, _D), lambda i: (i, 0)),
            pl.BlockSpec((_D, _D), lambda i: (0, 0)),
            pl.BlockSpec((1, _D), lambda i: (0, 0)),
        ],
        out_specs=pl.BlockSpec((_BM, _D), lambda i: (i, 0)),
        out_shape=jax.ShapeDtypeStruct((m, _D), jnp.float32),
    )(x, w, b)


def _layer_tc(parts, h0, w, beta):
    m = h0.shape[0]

    def body(p_ref, h0_ref, w_ref, o_ref):
        sup = ((1.0 - _ALPHA) * (p_ref[0] + p_ref[1])
               + _ALPHA * h0_ref[...])
        t = jnp.dot(sup, w_ref[...], preferred_element_type=jnp.float32)
        o_ref[...] = jnp.maximum((1.0 - beta) * sup + beta * t, 0.0)

    return pl.pallas_call(
        body,
        grid=(m // _BM,),
        in_specs=[
            pl.BlockSpec((2, _BM, _D), lambda i: (0, i, 0)),
            pl.BlockSpec((_BM, _D), lambda i: (i, 0)),
            pl.BlockSpec((_D, _D), lambda i: (0, 0)),
        ],
        out_specs=pl.BlockSpec((_BM, _D), lambda i: (i, 0)),
        out_shape=jax.ShapeDtypeStruct((m, _D), jnp.float32),
    )(parts, h0, w)


def _final_tc(parts, h0, wc, beta, w1, b1):
    m = h0.shape[0]
    d_out = w1.shape[1]

    def body(p_ref, h0_ref, wc_ref, w1_ref, b1_ref, o_ref):
        sup = ((1.0 - _ALPHA) * (p_ref[0] + p_ref[1])
               + _ALPHA * h0_ref[...])
        t = jnp.dot(sup, wc_ref[...], preferred_element_type=jnp.float32)
        h = jnp.maximum((1.0 - beta) * sup + beta * t, 0.0)
        logits = jnp.dot(h, w1_ref[...],
                         preferred_element_type=jnp.float32) + b1_ref[...]
        mx = jnp.max(logits, axis=1, keepdims=True)
        lse = jnp.log(jnp.sum(jnp.exp(logits - mx), axis=1, keepdims=True))
        o_ref[...] = logits - mx - lse

    return pl.pallas_call(
        body,
        grid=(m // _BM,),
        in_specs=[
            pl.BlockSpec((2, _BM, _D), lambda i: (0, i, 0)),
            pl.BlockSpec((_BM, _D), lambda i: (i, 0)),
            pl.BlockSpec((_D, _D), lambda i: (0, 0)),
            pl.BlockSpec((_D, d_out), lambda i: (0, 0)),
            pl.BlockSpec((1, d_out), lambda i: (0, 0)),
        ],
        out_specs=pl.BlockSpec((_BM, d_out), lambda i: (i, 0)),
        out_shape=jax.ShapeDtypeStruct((m, d_out), jnp.float32),
    )(parts, h0, wc, w1, b1)


def kernel(feature, edge_weight, W_fc0, b_fc0, W_conv, W_fc1, b_fc1,
           edge_index):
    n = feature.shape[0]
    e = edge_index.shape[1]

    per_chunk = _NW * _CH
    nch = -(-e // per_chunk)
    e_pad = nch * per_chunk
    ei = edge_index.astype(jnp.int32)
    dst3 = jnp.pad(ei[0], (0, e_pad - e)).reshape(_NW, nch, _CH)
    src3 = jnp.pad(ei[1], (0, e_pad - e)).reshape(_NW, nch, _CH)
    w3 = jnp.pad(edge_weight, (0, e_pad - e)).reshape(_NW, nch, _CH)

    h0 = _mlp0_tc(feature, W_fc0, b_fc0.reshape(1, _D))
    h = h0
    out = None
    for l in range(1, _L + 1):
        beta = math.log(_LAMDA / l + 1.0)
        parts = _spmm_sc(h, src3, dst3, w3, n)
        if l < _L:
            h = _layer_tc(parts, h0, W_conv[l - 1], beta)
        else:
            out = _final_tc(parts, h0, W_conv[l - 1], beta, W_fc1,
                            b_fc1.reshape(1, -1))
    return out
